# baseline (speedup 1.0000x reference)
import functools

import jax
import jax.numpy as jnp
from jax.experimental import pallas as pl
from jax.experimental.pallas import tpu as pltpu


def _round_up(x, m):
    return (x + m - 1) // m * m


def _fused_rows_kernel(x_ref, wi0_ref, bi0_ref, wir_ref, bir_ref, bh_ref,
                       wo_ref, bo_ref, out_ref, lp_ref, *, num_rest, seq):
    h = jnp.tanh(
        jnp.dot(x_ref[...], wi0_ref[...],
                preferred_element_type=jnp.float32)
        + (bi0_ref[...] + bh_ref[0]))
    for j in range(num_rest):
        h = jnp.tanh(
            jnp.dot(h, wir_ref[j], preferred_element_type=jnp.float32)
            + (bir_ref[j] + bh_ref[j + 1]))
    out_ref[...] = h

    tm, hp = h.shape
    nb = tm // seq
    last = h.reshape(nb, seq, hp)[:, seq - 1, :]
    logits = jnp.dot(last, wo_ref[...],
                     preferred_element_type=jnp.float32) + bo_ref[...]
    m = jnp.max(logits, axis=-1, keepdims=True)
    e = logits - m
    lp_ref[...] = e - jnp.log(jnp.sum(jnp.exp(e), axis=-1, keepdims=True))


def _rows_only_kernel(x_ref, wi0_ref, bi0_ref, wir_ref, bir_ref, bh_ref,
                      out_ref, *, num_rest):
    h = jnp.tanh(
        jnp.dot(x_ref[...], wi0_ref[...],
                preferred_element_type=jnp.float32)
        + (bi0_ref[...] + bh_ref[0]))
    for j in range(num_rest):
        h = jnp.tanh(
            jnp.dot(h, wir_ref[j], preferred_element_type=jnp.float32)
            + (bir_ref[j] + bh_ref[j + 1]))
    out_ref[...] = h


def _head_kernel(h_ref, wo_ref, bo_ref, lp_ref):
    last = h_ref[:, 0, :]
    logits = jnp.dot(last, wo_ref[...],
                     preferred_element_type=jnp.float32) + bo_ref[...]
    m = jnp.max(logits, axis=-1, keepdims=True)
    e = logits - m
    lp_ref[...] = e - jnp.log(jnp.sum(jnp.exp(e), axis=-1, keepdims=True))


@jax.jit
def _forward(x, wi0, bi0, wir, bir, bh, wo, bo):
    batch, seq, d_in = x.shape
    hidden = wi0.shape[-1]
    l_rest = wir.shape[0]
    out_size = wo.shape[-1]

    hp = _round_up(hidden, 128)
    op = _round_up(out_size, 128)
    if hp != hidden:
        wi0 = jnp.pad(wi0, ((0, 0), (0, hp - hidden)))
        bi0 = jnp.pad(bi0, ((0, 0), (0, hp - hidden)))
        wir = jnp.pad(wir, ((0, 0), (0, hp - hidden), (0, hp - hidden)))
        bir = jnp.pad(bir, ((0, 0), (0, 0), (0, hp - hidden)))
        bh = jnp.pad(bh, ((0, 0), (0, 0), (0, hp - hidden)))
        wo = jnp.pad(wo, ((0, hp - hidden), (0, 0)))
    if op != out_size:
        wo = jnp.pad(wo, ((0, 0), (0, op - out_size)))
        bo = jnp.pad(bo, ((0, 0), (0, op - out_size)),
                     constant_values=-jnp.inf)
    if l_rest == 0:
        wir = jnp.zeros((1, hp, hp), jnp.float32)
        bir = jnp.zeros((1, 1, hp), jnp.float32)
    l_eff = wir.shape[0]

    rows = batch * seq
    x_rows = x.reshape(rows, d_in)

    nb = max(1, min(batch, 512 // seq if seq <= 512 else 1))
    while batch % nb != 0:
        nb -= 1
    tm = nb * seq
    fuse_head = (tm % seq == 0) and (rows % tm == 0) and (rows // tm >= 2)

    vmem_limit = int(min(128 * 1024 * 1024, 2 * (
        2 * tm * d_in * 4 + 2 * tm * hp * 4 + 2 * nb * op * 4
        + d_in * hp * 4 + l_eff * hp * hp * 4 + hp * op * 4
        + (2 + 2 * l_eff) * hp * 4 + op * 4)))
    cost = pl.CostEstimate(
        flops=2 * rows * (d_in + l_rest * hp) * hp + 2 * batch * hp * op,
        transcendentals=rows * hp * (1 + l_rest) + batch * op,
        bytes_accessed=(rows * d_in * 4 + rows * hp * 4 + batch * op * 4
                        + d_in * hp * 4 + l_eff * hp * hp * 4 + hp * op * 4))

    def w_spec(shape, index_map):
        return pl.BlockSpec(shape, index_map, pipeline_mode=pl.Buffered(1))

    base_specs = [
        pl.BlockSpec((tm, d_in), lambda i: (i, 0)),
        w_spec((d_in, hp), lambda i: (0, 0)),
        w_spec((1, hp), lambda i: (0, 0)),
        w_spec((l_eff, hp, hp), lambda i: (0, 0, 0)),
        w_spec((l_eff, 1, hp), lambda i: (0, 0, 0)),
        w_spec((bh.shape[0], 1, hp), lambda i: (0, 0, 0)),
    ]

    if fuse_head:
        kfn = functools.partial(_fused_rows_kernel, num_rest=l_rest, seq=seq)
        h_rows, lp = pl.pallas_call(
            kfn,
            out_shape=(jax.ShapeDtypeStruct((rows, hp), jnp.float32),
                       jax.ShapeDtypeStruct((batch, op), jnp.float32)),
            grid=(rows // tm,),
            in_specs=base_specs + [
                w_spec((hp, op), lambda i: (0, 0)),
                w_spec((1, op), lambda i: (0, 0)),
            ],
            out_specs=(pl.BlockSpec((tm, hp), lambda i: (i, 0)),
                       pl.BlockSpec((nb, op), lambda i: (i, 0))),
            compiler_params=pltpu.CompilerParams(
                dimension_semantics=("parallel",),
                vmem_limit_bytes=vmem_limit),
            cost_estimate=cost,
        )(x_rows, wi0, bi0, wir, bir, bh, wo, bo)
    else:
        kfn = functools.partial(_rows_only_kernel, num_rest=l_rest)
        h_rows = pl.pallas_call(
            kfn,
            out_shape=jax.ShapeDtypeStruct((rows, hp), jnp.float32),
            grid=(pl.cdiv(rows, tm),),
            in_specs=base_specs,
            out_specs=pl.BlockSpec((tm, hp), lambda i: (i, 0)),
            compiler_params=pltpu.CompilerParams(
                dimension_semantics=("parallel",),
                vmem_limit_bytes=vmem_limit),
            cost_estimate=cost,
        )(x_rows, wi0, bi0, wir, bir, bh)
        h3 = h_rows.reshape(batch, seq, hp)
        lp = pl.pallas_call(
            _head_kernel,
            out_shape=jax.ShapeDtypeStruct((batch, op), jnp.float32),
            grid=(1,),
            in_specs=[
                pl.BlockSpec((batch, 1, hp), lambda i: (0, seq - 1, 0)),
                pl.BlockSpec((hp, op), lambda i: (0, 0)),
                pl.BlockSpec((1, op), lambda i: (0, 0)),
            ],
            out_specs=pl.BlockSpec((batch, op), lambda i: (0, 0)),
        )(h3, wo, bo)

    out3 = h_rows.reshape(batch, seq, hp)
    outputs = out3[..., :hidden] if hp != hidden else out3
    log_probs = lp[:, :out_size] if op != out_size else lp
    return log_probs, outputs


def kernel(x, wi0, bi0, wir, bir, wh, bh, wo, bo):
    return _forward(x, wi0, bi0, wir, bir, bh, wo, bo)


# speedup vs baseline: 1.1410x; 1.1410x over previous
import functools

import jax
import jax.numpy as jnp
from jax.experimental import pallas as pl
from jax.experimental.pallas import tpu as pltpu


def _round_up(x, m):
    return (x + m - 1) // m * m


def _fused_rows_kernel(x_ref, wi0_ref, bi0_ref, wir_ref, bir_ref, bh_ref,
                       wo_ref, bo_ref, out_ref, lp_ref, *, num_rest, seq):
    h = jnp.tanh(
        jnp.dot(x_ref[...], wi0_ref[...],
                preferred_element_type=jnp.float32)
        + (bi0_ref[...] + bh_ref[0]))
    for j in range(num_rest):
        h = jnp.tanh(
            jnp.dot(h, wir_ref[j], preferred_element_type=jnp.float32)
            + (bir_ref[j] + bh_ref[j + 1]))
    out_ref[...] = h

    tm, hp = h.shape
    nb = tm // seq
    last = h.reshape(nb, seq, hp)[:, seq - 1, :]
    logits = jnp.dot(last, wo_ref[...],
                     preferred_element_type=jnp.float32) + bo_ref[...]
    m = jnp.max(logits, axis=-1, keepdims=True)
    e = logits - m
    lp_ref[...] = e - jnp.log(jnp.sum(jnp.exp(e), axis=-1, keepdims=True))


def _rows_only_kernel(x_ref, wi0_ref, bi0_ref, wir_ref, bir_ref, bh_ref,
                      out_ref, *, num_rest):
    h = jnp.tanh(
        jnp.dot(x_ref[...], wi0_ref[...],
                preferred_element_type=jnp.float32)
        + (bi0_ref[...] + bh_ref[0]))
    for j in range(num_rest):
        h = jnp.tanh(
            jnp.dot(h, wir_ref[j], preferred_element_type=jnp.float32)
            + (bir_ref[j] + bh_ref[j + 1]))
    out_ref[...] = h


def _head_kernel(h_ref, wo_ref, bo_ref, lp_ref):
    last = h_ref[:, 0, :]
    logits = jnp.dot(last, wo_ref[...],
                     preferred_element_type=jnp.float32) + bo_ref[...]
    m = jnp.max(logits, axis=-1, keepdims=True)
    e = logits - m
    lp_ref[...] = e - jnp.log(jnp.sum(jnp.exp(e), axis=-1, keepdims=True))


@jax.jit
def _forward(x, wi0, bi0, wir, bir, bh, wo, bo):
    batch, seq, d_in = x.shape
    hidden = wi0.shape[-1]
    l_rest = wir.shape[0]
    out_size = wo.shape[-1]

    hp = _round_up(hidden, 128)
    op = _round_up(out_size, 128)
    if hp != hidden:
        wi0 = jnp.pad(wi0, ((0, 0), (0, hp - hidden)))
        bi0 = jnp.pad(bi0, ((0, 0), (0, hp - hidden)))
        wir = jnp.pad(wir, ((0, 0), (0, hp - hidden), (0, hp - hidden)))
        bir = jnp.pad(bir, ((0, 0), (0, 0), (0, hp - hidden)))
        bh = jnp.pad(bh, ((0, 0), (0, 0), (0, hp - hidden)))
        wo = jnp.pad(wo, ((0, hp - hidden), (0, 0)))
    if op != out_size:
        wo = jnp.pad(wo, ((0, 0), (0, op - out_size)))
        bo = jnp.pad(bo, ((0, 0), (0, op - out_size)),
                     constant_values=-jnp.inf)
    if l_rest == 0:
        wir = jnp.zeros((1, hp, hp), jnp.float32)
        bir = jnp.zeros((1, 1, hp), jnp.float32)
    l_eff = wir.shape[0]

    rows = batch * seq
    x_rows = x.reshape(rows, d_in)

    nb = max(1, min(batch, 1024 // seq if seq <= 1024 else 1))
    while batch % nb != 0:
        nb -= 1
    tm = nb * seq
    fuse_head = (tm % seq == 0) and (rows % tm == 0) and (rows // tm >= 2)

    vmem_limit = int(min(128 * 1024 * 1024, 2 * (
        2 * tm * d_in * 4 + 2 * tm * hp * 4 + 2 * nb * op * 4
        + d_in * hp * 4 + l_eff * hp * hp * 4 + hp * op * 4
        + (2 + 2 * l_eff) * hp * 4 + op * 4)))
    cost = pl.CostEstimate(
        flops=2 * rows * (d_in + l_rest * hp) * hp + 2 * batch * hp * op,
        transcendentals=rows * hp * (1 + l_rest) + batch * op,
        bytes_accessed=(rows * d_in * 4 + rows * hp * 4 + batch * op * 4
                        + d_in * hp * 4 + l_eff * hp * hp * 4 + hp * op * 4))

    def w_spec(shape, index_map):
        return pl.BlockSpec(shape, index_map, pipeline_mode=pl.Buffered(1))

    base_specs = [
        pl.BlockSpec((tm, d_in), lambda i: (i, 0)),
        w_spec((d_in, hp), lambda i: (0, 0)),
        w_spec((1, hp), lambda i: (0, 0)),
        w_spec((l_eff, hp, hp), lambda i: (0, 0, 0)),
        w_spec((l_eff, 1, hp), lambda i: (0, 0, 0)),
        w_spec((bh.shape[0], 1, hp), lambda i: (0, 0, 0)),
    ]

    if fuse_head:
        kfn = functools.partial(_fused_rows_kernel, num_rest=l_rest, seq=seq)
        h_rows, lp = pl.pallas_call(
            kfn,
            out_shape=(jax.ShapeDtypeStruct((rows, hp), jnp.float32),
                       jax.ShapeDtypeStruct((batch, op), jnp.float32)),
            grid=(rows // tm,),
            in_specs=base_specs + [
                w_spec((hp, op), lambda i: (0, 0)),
                w_spec((1, op), lambda i: (0, 0)),
            ],
            out_specs=(pl.BlockSpec((tm, hp), lambda i: (i, 0)),
                       pl.BlockSpec((nb, op), lambda i: (i, 0))),
            compiler_params=pltpu.CompilerParams(
                dimension_semantics=("parallel",),
                vmem_limit_bytes=vmem_limit),
            cost_estimate=cost,
        )(x_rows, wi0, bi0, wir, bir, bh, wo, bo)
    else:
        kfn = functools.partial(_rows_only_kernel, num_rest=l_rest)
        h_rows = pl.pallas_call(
            kfn,
            out_shape=jax.ShapeDtypeStruct((rows, hp), jnp.float32),
            grid=(pl.cdiv(rows, tm),),
            in_specs=base_specs,
            out_specs=pl.BlockSpec((tm, hp), lambda i: (i, 0)),
            compiler_params=pltpu.CompilerParams(
                dimension_semantics=("parallel",),
                vmem_limit_bytes=vmem_limit),
            cost_estimate=cost,
        )(x_rows, wi0, bi0, wir, bir, bh)
        h3 = h_rows.reshape(batch, seq, hp)
        lp = pl.pallas_call(
            _head_kernel,
            out_shape=jax.ShapeDtypeStruct((batch, op), jnp.float32),
            grid=(1,),
            in_specs=[
                pl.BlockSpec((batch, 1, hp), lambda i: (0, seq - 1, 0)),
                pl.BlockSpec((hp, op), lambda i: (0, 0)),
                pl.BlockSpec((1, op), lambda i: (0, 0)),
            ],
            out_specs=pl.BlockSpec((batch, op), lambda i: (0, 0)),
        )(h3, wo, bo)

    out3 = h_rows.reshape(batch, seq, hp)
    outputs = out3[..., :hidden] if hp != hidden else out3
    log_probs = lp[:, :out_size] if op != out_size else lp
    return log_probs, outputs


def kernel(x, wi0, bi0, wir, bir, wh, bh, wo, bo):
    return _forward(x, wi0, bi0, wir, bir, bh, wo, bo)


# speedup vs baseline: 1.2135x; 1.0635x over previous
import functools

import jax
import jax.numpy as jnp
from jax.experimental import pallas as pl
from jax.experimental.pallas import tpu as pltpu


def _round_up(x, m):
    return (x + m - 1) // m * m


def _fused_rows_kernel(x_ref, wi0_ref, bi0_ref, wir_ref, bir_ref, bh_ref,
                       wo_ref, bo_ref, out_ref, lp_ref, *, num_rest, seq):
    h = jnp.tanh(
        jnp.dot(x_ref[...], wi0_ref[...],
                preferred_element_type=jnp.float32)
        + (bi0_ref[...] + bh_ref[0]))
    for j in range(num_rest):
        h = jnp.tanh(
            jnp.dot(h, wir_ref[j], preferred_element_type=jnp.float32)
            + (bir_ref[j] + bh_ref[j + 1]))
    out_ref[...] = h

    tm, hp = h.shape
    nb = tm // seq
    last = h.reshape(nb, seq, hp)[:, seq - 1, :]
    logits = jnp.dot(last, wo_ref[...],
                     preferred_element_type=jnp.float32) + bo_ref[...]
    m = jnp.max(logits, axis=-1, keepdims=True)
    e = logits - m
    lp_ref[...] = e - jnp.log(jnp.sum(jnp.exp(e), axis=-1, keepdims=True))


def _rows_only_kernel(x_ref, wi0_ref, bi0_ref, wir_ref, bir_ref, bh_ref,
                      out_ref, *, num_rest):
    h = jnp.tanh(
        jnp.dot(x_ref[...], wi0_ref[...],
                preferred_element_type=jnp.float32)
        + (bi0_ref[...] + bh_ref[0]))
    for j in range(num_rest):
        h = jnp.tanh(
            jnp.dot(h, wir_ref[j], preferred_element_type=jnp.float32)
            + (bir_ref[j] + bh_ref[j + 1]))
    out_ref[...] = h


def _head_kernel(h_ref, wo_ref, bo_ref, lp_ref):
    last = h_ref[:, 0, :]
    logits = jnp.dot(last, wo_ref[...],
                     preferred_element_type=jnp.float32) + bo_ref[...]
    m = jnp.max(logits, axis=-1, keepdims=True)
    e = logits - m
    lp_ref[...] = e - jnp.log(jnp.sum(jnp.exp(e), axis=-1, keepdims=True))


@jax.jit
def _forward(x, wi0, bi0, wir, bir, bh, wo, bo):
    batch, seq, d_in = x.shape
    hidden = wi0.shape[-1]
    l_rest = wir.shape[0]
    out_size = wo.shape[-1]

    hp = _round_up(hidden, 128)
    op = _round_up(out_size, 128)
    if hp != hidden:
        wi0 = jnp.pad(wi0, ((0, 0), (0, hp - hidden)))
        bi0 = jnp.pad(bi0, ((0, 0), (0, hp - hidden)))
        wir = jnp.pad(wir, ((0, 0), (0, hp - hidden), (0, hp - hidden)))
        bir = jnp.pad(bir, ((0, 0), (0, 0), (0, hp - hidden)))
        bh = jnp.pad(bh, ((0, 0), (0, 0), (0, hp - hidden)))
        wo = jnp.pad(wo, ((0, hp - hidden), (0, 0)))
    if op != out_size:
        wo = jnp.pad(wo, ((0, 0), (0, op - out_size)))
        bo = jnp.pad(bo, ((0, 0), (0, op - out_size)),
                     constant_values=-jnp.inf)
    if l_rest == 0:
        wir = jnp.zeros((1, hp, hp), jnp.float32)
        bir = jnp.zeros((1, 1, hp), jnp.float32)
    l_eff = wir.shape[0]

    rows = batch * seq
    x_rows = x.reshape(rows, d_in)

    nb = max(1, min(batch, 2048 // seq if seq <= 2048 else 1))
    while batch % nb != 0:
        nb -= 1
    tm = nb * seq
    fuse_head = (tm % seq == 0) and (rows % tm == 0) and (rows // tm >= 2)

    vmem_limit = int(min(128 * 1024 * 1024, 2 * (
        2 * tm * d_in * 4 + 2 * tm * hp * 4 + 2 * nb * op * 4
        + d_in * hp * 4 + l_eff * hp * hp * 4 + hp * op * 4
        + (2 + 2 * l_eff) * hp * 4 + op * 4)))
    cost = pl.CostEstimate(
        flops=2 * rows * (d_in + l_rest * hp) * hp + 2 * batch * hp * op,
        transcendentals=rows * hp * (1 + l_rest) + batch * op,
        bytes_accessed=(rows * d_in * 4 + rows * hp * 4 + batch * op * 4
                        + d_in * hp * 4 + l_eff * hp * hp * 4 + hp * op * 4))

    def w_spec(shape, index_map):
        return pl.BlockSpec(shape, index_map, pipeline_mode=pl.Buffered(1))

    base_specs = [
        pl.BlockSpec((tm, d_in), lambda i: (i, 0)),
        w_spec((d_in, hp), lambda i: (0, 0)),
        w_spec((1, hp), lambda i: (0, 0)),
        w_spec((l_eff, hp, hp), lambda i: (0, 0, 0)),
        w_spec((l_eff, 1, hp), lambda i: (0, 0, 0)),
        w_spec((bh.shape[0], 1, hp), lambda i: (0, 0, 0)),
    ]

    if fuse_head:
        kfn = functools.partial(_fused_rows_kernel, num_rest=l_rest, seq=seq)
        h_rows, lp = pl.pallas_call(
            kfn,
            out_shape=(jax.ShapeDtypeStruct((rows, hp), jnp.float32),
                       jax.ShapeDtypeStruct((batch, op), jnp.float32)),
            grid=(rows // tm,),
            in_specs=base_specs + [
                w_spec((hp, op), lambda i: (0, 0)),
                w_spec((1, op), lambda i: (0, 0)),
            ],
            out_specs=(pl.BlockSpec((tm, hp), lambda i: (i, 0)),
                       pl.BlockSpec((nb, op), lambda i: (i, 0))),
            compiler_params=pltpu.CompilerParams(
                dimension_semantics=("parallel",),
                vmem_limit_bytes=vmem_limit),
            cost_estimate=cost,
        )(x_rows, wi0, bi0, wir, bir, bh, wo, bo)
    else:
        kfn = functools.partial(_rows_only_kernel, num_rest=l_rest)
        h_rows = pl.pallas_call(
            kfn,
            out_shape=jax.ShapeDtypeStruct((rows, hp), jnp.float32),
            grid=(pl.cdiv(rows, tm),),
            in_specs=base_specs,
            out_specs=pl.BlockSpec((tm, hp), lambda i: (i, 0)),
            compiler_params=pltpu.CompilerParams(
                dimension_semantics=("parallel",),
                vmem_limit_bytes=vmem_limit),
            cost_estimate=cost,
        )(x_rows, wi0, bi0, wir, bir, bh)
        h3 = h_rows.reshape(batch, seq, hp)
        lp = pl.pallas_call(
            _head_kernel,
            out_shape=jax.ShapeDtypeStruct((batch, op), jnp.float32),
            grid=(1,),
            in_specs=[
                pl.BlockSpec((batch, 1, hp), lambda i: (0, seq - 1, 0)),
                pl.BlockSpec((hp, op), lambda i: (0, 0)),
                pl.BlockSpec((1, op), lambda i: (0, 0)),
            ],
            out_specs=pl.BlockSpec((batch, op), lambda i: (0, 0)),
        )(h3, wo, bo)

    out3 = h_rows.reshape(batch, seq, hp)
    outputs = out3[..., :hidden] if hp != hidden else out3
    log_probs = lp[:, :out_size] if op != out_size else lp
    return log_probs, outputs


def kernel(x, wi0, bi0, wir, bir, wh, bh, wo, bo):
    return _forward(x, wi0, bi0, wir, bir, bh, wo, bo)


# speedup vs baseline: 1.5266x; 1.2581x over previous
import functools

import jax
import jax.numpy as jnp
from jax.experimental import pallas as pl
from jax.experimental.pallas import tpu as pltpu


def _round_up(x, m):
    return (x + m - 1) // m * m


def _fused_rows_kernel(x_ref, wi0_ref, bi0_ref, wir_ref, bir_ref, bh_ref,
                       wo_ref, bo_ref, out_ref, lp_ref, *, num_rest, seq):
    h = jnp.tanh(
        jnp.dot(x_ref[...], wi0_ref[...],
                preferred_element_type=jnp.float32)
        + (bi0_ref[...] + bh_ref[0]))
    for j in range(num_rest):
        h = jnp.tanh(
            jnp.dot(h, wir_ref[j], preferred_element_type=jnp.float32)
            + (bir_ref[j] + bh_ref[j + 1]))
    out_ref[...] = h

    tm, hp = h.shape
    nb = tm // seq
    last = h.reshape(nb, seq, hp)[:, seq - 1, :]
    logits = jnp.dot(last, wo_ref[...],
                     preferred_element_type=jnp.float32) + bo_ref[...]
    m = jnp.max(logits, axis=-1, keepdims=True)
    e = logits - m
    lp_ref[...] = e - jnp.log(jnp.sum(jnp.exp(e), axis=-1, keepdims=True))


def _rows_only_kernel(x_ref, wi0_ref, bi0_ref, wir_ref, bir_ref, bh_ref,
                      out_ref, *, num_rest):
    h = jnp.tanh(
        jnp.dot(x_ref[...], wi0_ref[...],
                preferred_element_type=jnp.float32)
        + (bi0_ref[...] + bh_ref[0]))
    for j in range(num_rest):
        h = jnp.tanh(
            jnp.dot(h, wir_ref[j], preferred_element_type=jnp.float32)
            + (bir_ref[j] + bh_ref[j + 1]))
    out_ref[...] = h


def _head_kernel(h_ref, wo_ref, bo_ref, lp_ref):
    last = h_ref[:, 0, :]
    logits = jnp.dot(last, wo_ref[...],
                     preferred_element_type=jnp.float32) + bo_ref[...]
    m = jnp.max(logits, axis=-1, keepdims=True)
    e = logits - m
    lp_ref[...] = e - jnp.log(jnp.sum(jnp.exp(e), axis=-1, keepdims=True))


@jax.jit
def _forward(x, wi0, bi0, wir, bir, bh, wo, bo):
    batch, seq, d_in = x.shape
    hidden = wi0.shape[-1]
    l_rest = wir.shape[0]
    out_size = wo.shape[-1]

    hp = _round_up(hidden, 128)
    op = _round_up(out_size, 128)
    if hp != hidden:
        wi0 = jnp.pad(wi0, ((0, 0), (0, hp - hidden)))
        bi0 = jnp.pad(bi0, ((0, 0), (0, hp - hidden)))
        wir = jnp.pad(wir, ((0, 0), (0, hp - hidden), (0, hp - hidden)))
        bir = jnp.pad(bir, ((0, 0), (0, 0), (0, hp - hidden)))
        bh = jnp.pad(bh, ((0, 0), (0, 0), (0, hp - hidden)))
        wo = jnp.pad(wo, ((0, hp - hidden), (0, 0)))
    if op != out_size:
        wo = jnp.pad(wo, ((0, 0), (0, op - out_size)))
        bo = jnp.pad(bo, ((0, 0), (0, op - out_size)),
                     constant_values=-jnp.inf)
    if l_rest == 0:
        wir = jnp.zeros((1, hp, hp), jnp.float32)
        bir = jnp.zeros((1, 1, hp), jnp.float32)
    l_eff = wir.shape[0]

    rows = batch * seq
    x_rows = x.reshape(rows, d_in)

    nb = max(1, min(batch, 4096 // seq if seq <= 4096 else 1))
    while batch % nb != 0:
        nb -= 1
    tm = nb * seq
    fuse_head = (tm % seq == 0) and (rows % tm == 0) and (rows // tm >= 2)

    vmem_limit = int(min(128 * 1024 * 1024, 2 * (
        2 * tm * d_in * 4 + 2 * tm * hp * 4 + 2 * nb * op * 4
        + d_in * hp * 4 + l_eff * hp * hp * 4 + hp * op * 4
        + (2 + 2 * l_eff) * hp * 4 + op * 4)))
    cost = pl.CostEstimate(
        flops=2 * rows * (d_in + l_rest * hp) * hp + 2 * batch * hp * op,
        transcendentals=rows * hp * (1 + l_rest) + batch * op,
        bytes_accessed=(rows * d_in * 4 + rows * hp * 4 + batch * op * 4
                        + d_in * hp * 4 + l_eff * hp * hp * 4 + hp * op * 4))

    def w_spec(shape, index_map):
        return pl.BlockSpec(shape, index_map, pipeline_mode=pl.Buffered(1))

    base_specs = [
        pl.BlockSpec((tm, d_in), lambda i: (i, 0)),
        w_spec((d_in, hp), lambda i: (0, 0)),
        w_spec((1, hp), lambda i: (0, 0)),
        w_spec((l_eff, hp, hp), lambda i: (0, 0, 0)),
        w_spec((l_eff, 1, hp), lambda i: (0, 0, 0)),
        w_spec((bh.shape[0], 1, hp), lambda i: (0, 0, 0)),
    ]

    if fuse_head:
        kfn = functools.partial(_fused_rows_kernel, num_rest=l_rest, seq=seq)
        h_rows, lp = pl.pallas_call(
            kfn,
            out_shape=(jax.ShapeDtypeStruct((rows, hp), jnp.float32),
                       jax.ShapeDtypeStruct((batch, op), jnp.float32)),
            grid=(rows // tm,),
            in_specs=base_specs + [
                w_spec((hp, op), lambda i: (0, 0)),
                w_spec((1, op), lambda i: (0, 0)),
            ],
            out_specs=(pl.BlockSpec((tm, hp), lambda i: (i, 0)),
                       pl.BlockSpec((nb, op), lambda i: (i, 0))),
            compiler_params=pltpu.CompilerParams(
                dimension_semantics=("parallel",),
                vmem_limit_bytes=vmem_limit),
            cost_estimate=cost,
        )(x_rows, wi0, bi0, wir, bir, bh, wo, bo)
    else:
        kfn = functools.partial(_rows_only_kernel, num_rest=l_rest)
        h_rows = pl.pallas_call(
            kfn,
            out_shape=jax.ShapeDtypeStruct((rows, hp), jnp.float32),
            grid=(pl.cdiv(rows, tm),),
            in_specs=base_specs,
            out_specs=pl.BlockSpec((tm, hp), lambda i: (i, 0)),
            compiler_params=pltpu.CompilerParams(
                dimension_semantics=("parallel",),
                vmem_limit_bytes=vmem_limit),
            cost_estimate=cost,
        )(x_rows, wi0, bi0, wir, bir, bh)
        h3 = h_rows.reshape(batch, seq, hp)
        lp = pl.pallas_call(
            _head_kernel,
            out_shape=jax.ShapeDtypeStruct((batch, op), jnp.float32),
            grid=(1,),
            in_specs=[
                pl.BlockSpec((batch, 1, hp), lambda i: (0, seq - 1, 0)),
                pl.BlockSpec((hp, op), lambda i: (0, 0)),
                pl.BlockSpec((1, op), lambda i: (0, 0)),
            ],
            out_specs=pl.BlockSpec((batch, op), lambda i: (0, 0)),
        )(h3, wo, bo)

    out3 = h_rows.reshape(batch, seq, hp)
    outputs = out3[..., :hidden] if hp != hidden else out3
    log_probs = lp[:, :out_size] if op != out_size else lp
    return log_probs, outputs


def kernel(x, wi0, bi0, wir, bir, wh, bh, wo, bo):
    return _forward(x, wi0, bi0, wir, bir, bh, wo, bo)
